# baseline (speedup 1.0000x reference)
import functools

import jax
import jax.numpy as jnp
from jax.experimental import pallas as pl


NUM_EMB = 1024
EMB_DIM = 64
BATCH_PER_STEP = 4


def _vq_kernel(nbatch, x_ref, cbf_ref, q_bnc_ref, q_bcn_ref, loss_ref):
    step = pl.program_id(0)
    cbf = cbf_ref[...]
    e_sq = jnp.sum(cbf * cbf, axis=-1)
    iota_k = jax.lax.broadcasted_iota(jnp.int32, (1, NUM_EMB), 1)
    loss = jnp.zeros((1, 1), jnp.float32)
    for b in range(nbatch):
        xc = x_ref[b]
        xp = xc.T
        x_sq = jnp.sum(xp * xp, axis=-1, keepdims=True)
        dot2 = jnp.dot(xp + xp, cbf.T, preferred_element_type=jnp.float32)
        d2 = (x_sq + e_sq[None, :]) - dot2
        dis = jnp.sqrt(jnp.maximum(d2, 0.0))
        zf = jnp.argmin(dis, axis=-1)
        onehot = jnp.where(zf[:, None] == iota_k, 1.0, 0.0).astype(jnp.float32)
        q = jnp.dot(onehot, cbf, preferred_element_type=jnp.float32)
        q_bnc_ref[b] = q
        qq = xp + (q - xp)
        q_bcn_ref[b] = qq.T
        diff = xp - q
        loss = loss + jnp.sum(diff * diff).reshape(1, 1)

    @pl.when(step == 0)
    def _():
        loss_ref[...] = jnp.zeros((1, 1), jnp.float32)
    loss_ref[...] += loss


@functools.partial(jax.jit, static_argnames=())
def kernel(x, codebook):
    B, C, H, W = x.shape
    N = H * W
    x3 = x.reshape(B, C, N)
    cbf = codebook.reshape(-1, 128, EMB_DIM)[:, ::-1, :].reshape(-1, EMB_DIM)
    bps = BATCH_PER_STEP
    nsteps = B // bps

    q_bnc, q_bcn, loss_sum = pl.pallas_call(
        functools.partial(_vq_kernel, bps),
        grid=(nsteps,),
        in_specs=[
            pl.BlockSpec((bps, C, N), lambda i: (i, 0, 0)),
            pl.BlockSpec((NUM_EMB, EMB_DIM), lambda i: (0, 0)),
        ],
        out_specs=[
            pl.BlockSpec((bps, N, C), lambda i: (i, 0, 0)),
            pl.BlockSpec((bps, C, N), lambda i: (i, 0, 0)),
            pl.BlockSpec((1, 1), lambda i: (0, 0)),
        ],
        out_shape=[
            jax.ShapeDtypeStruct((B, N, C), jnp.float32),
            jax.ShapeDtypeStruct((B, C, N), jnp.float32),
            jax.ShapeDtypeStruct((1, 1), jnp.float32),
        ],
    )(x3, cbf)

    n_elems = jnp.float32(B * N * C)
    commitment_loss = (loss_sum[0, 0] / n_elems).astype(jnp.float32)
    codebook_loss = commitment_loss
    quantizer_loss = jnp.float32(0.2) * commitment_loss + codebook_loss

    quantized = q_bcn.reshape(B, C, H, W)
    min_index_r = q_bnc.reshape(B, C, H, W)
    return (quantized, codebook_loss, commitment_loss, quantizer_loss, min_index_r)


# speedup vs baseline: 1.0912x; 1.0912x over previous
import functools

import jax
import jax.numpy as jnp
from jax.experimental import pallas as pl


NUM_EMB = 1024
EMB_DIM = 64
BATCH_PER_STEP = 4


def _vq_kernel(nbatch, x_ref, cb_ref, q_bnc_ref, q_bcn_ref, loss_ref):
    step = pl.program_id(0)
    cb = cb_ref[...]
    e_sq = jnp.sum(cb * cb, axis=-1)
    iota_k = jax.lax.broadcasted_iota(jnp.int32, (1, NUM_EMB), 1)
    loss = jnp.zeros((1, 1), jnp.float32)
    for b in range(nbatch):
        xc = x_ref[b]
        xp = xc.T
        x_sq = jnp.sum(xp * xp, axis=-1, keepdims=True)
        dot2 = jnp.dot(xp + xp, cb.T, preferred_element_type=jnp.float32)
        d2 = (x_sq + e_sq[None, :]) - dot2
        m2 = jnp.min(d2, axis=-1, keepdims=True)
        m = jnp.sqrt(jnp.maximum(m2, 0.0))
        cbits = jax.lax.bitcast_convert_type(m * m, jnp.int32)
        offs = jax.lax.broadcasted_iota(jnp.int32, (1, 24), 1) - 8
        pb = jnp.maximum(cbits + offs, 0)
        px = jax.lax.bitcast_convert_type(pb, jnp.float32)
        hits = jnp.sqrt(px) == m
        B = jnp.max(jnp.where(hits, px, jnp.float32(-1.0)),
                    axis=-1, keepdims=True)
        cand = jnp.where(d2 <= B, iota_k, NUM_EMB)
        z = jnp.min(cand, axis=-1)
        onehot = jnp.where(z[:, None] == iota_k, 1.0, 0.0)
        q = jnp.dot(onehot, cb, preferred_element_type=jnp.float32)
        q_bnc_ref[b] = q
        qq = xp + (q - xp)
        q_bcn_ref[b] = qq.T
        diff = xp - q
        loss = loss + jnp.sum(diff * diff).reshape(1, 1)

    @pl.when(step == 0)
    def _():
        loss_ref[...] = jnp.zeros((1, 1), jnp.float32)
    loss_ref[...] += loss


@functools.partial(jax.jit, static_argnames=())
def kernel(x, codebook):
    B, C, H, W = x.shape
    N = H * W
    x3 = x.reshape(B, C, N)
    bps = BATCH_PER_STEP
    nsteps = B // bps

    q_bnc, q_bcn, loss_sum = pl.pallas_call(
        functools.partial(_vq_kernel, bps),
        grid=(nsteps,),
        in_specs=[
            pl.BlockSpec((bps, C, N), lambda i: (i, 0, 0)),
            pl.BlockSpec((NUM_EMB, EMB_DIM), lambda i: (0, 0)),
        ],
        out_specs=[
            pl.BlockSpec((bps, N, C), lambda i: (i, 0, 0)),
            pl.BlockSpec((bps, C, N), lambda i: (i, 0, 0)),
            pl.BlockSpec((1, 1), lambda i: (0, 0)),
        ],
        out_shape=[
            jax.ShapeDtypeStruct((B, N, C), jnp.float32),
            jax.ShapeDtypeStruct((B, C, N), jnp.float32),
            jax.ShapeDtypeStruct((1, 1), jnp.float32),
        ],
    )(x3, codebook)

    n_elems = jnp.float32(B * N * C)
    commitment_loss = (loss_sum[0, 0] / n_elems).astype(jnp.float32)
    codebook_loss = commitment_loss
    quantizer_loss = jnp.float32(0.2) * commitment_loss + codebook_loss

    quantized = q_bcn.reshape(B, C, H, W)
    min_index_r = q_bnc.reshape(B, C, H, W)
    return (quantized, codebook_loss, commitment_loss, quantizer_loss, min_index_r)
